# Initial kernel scaffold; baseline (speedup 1.0000x reference)
import jax
import jax.numpy as jnp
from jax.experimental import pallas as pl


def kernel(x, state, tau1, scale1, tau2, scale2, Win, bin_, rec_w, pos_bias, Wout, bout, Wr, W1, b1, W2, b2):
    raise NotImplementedError("write your pallas kernel here")


# speedup vs baseline: 1.3816x; 1.3816x over previous
import functools

import jax
import jax.numpy as jnp
from jax.experimental import pallas as pl
from jax.experimental.pallas import tpu as pltpu

DM = 1024
DS = 64
NE = 8
NA = 2
FH = 2048
CK = 64
EPS = 1e-8


def _compnorm(x, tau, scale):
    rms = jax.lax.rsqrt(jnp.mean(x * x, axis=-1, keepdims=True) + EPS)
    x_norm = x * rms
    xc = x - jnp.mean(x, axis=-1, keepdims=True)
    gate = jax.nn.softmax(xc / jnp.maximum(tau, 1.0), axis=-1)
    return x_norm * gate * scale * DM


def _k1(x_ref, tau_ref, scale_ref, wa_ref, wg_ref, ba_ref, bg_ref, u_ref):
    x = x_ref[...]
    h = _compnorm(x, tau_ref[0, 0], scale_ref[...])
    iv = jnp.tanh(jnp.dot(h, wa_ref[...], preferred_element_type=jnp.float32)
                  + ba_ref[...])
    g = jax.nn.sigmoid(jnp.dot(h, wg_ref[...], preferred_element_type=jnp.float32)
                       + bg_ref[...])
    u_ref[...] = g * iv


def _k2(ut_ref, u_ref, rec_ref, pos_ref, state_ref,
        fu_ref, hs_ref, ns_ref, r_scr, *, nb):
    la_col = jnp.log(jax.nn.sigmoid(rec_ref[...] + pos_ref[...]))
    la_row = la_col.reshape(1, DS)
    i_col = jax.lax.broadcasted_iota(jnp.int32, (CK, 1), 0).astype(jnp.float32)
    j2 = jax.lax.broadcasted_iota(jnp.int32, (CK, CK), 0).astype(jnp.float32)
    i2 = jax.lax.broadcasted_iota(jnp.int32, (CK, CK), 1).astype(jnp.float32)
    expn = jnp.maximum(j2 - i2, 0.0)
    mask = (j2 >= i2).astype(jnp.float32)
    L = jnp.exp(la_col.reshape(DS, 1, 1) * expn[None]) * mask[None]
    fu = jax.lax.dot_general(L, ut_ref[...],
                             (((2,), (1,)), ((0,), (0,))),
                             preferred_element_type=jnp.float32)
    fu_ref[...] = fu
    w = jnp.exp(la_row * (CK - 1.0 - i_col))
    r_scr[...] = jnp.sum(u_ref[...] * w[None], axis=1)
    a_ck = jnp.exp(la_row * float(CK))

    bsz = state_ref.shape[0]

    def body(c, h_cur):
        for b in range(bsz):
            hs_ref[pl.ds(b * nb + c, 1), :] = h_cur[b:b + 1]
        rows = jnp.concatenate(
            [r_scr[pl.ds(b * nb + c, 1), :] for b in range(bsz)], axis=0)
        return h_cur * a_ck + rows

    h_fin = jax.lax.fori_loop(0, nb, body, state_ref[...])
    ns_ref[...] = h_fin


def _k3(fu_ref, hs_ref, ap_ref, x_ref, tau_ref, scale_ref, wout_ref,
        bout_ref, wr_ref, x2_ref, h3_ref, wd_ref, ps_ref, cs_ref):
    states = fu_ref[...] + hs_ref[...] * ap_ref[...]
    h2 = jnp.dot(states, wout_ref[...], preferred_element_type=jnp.float32) \
        + bout_ref[...]
    x2 = x_ref[...] + h2
    x2_ref[...] = x2
    h3 = _compnorm(x2, tau_ref[0, 0], scale_ref[...])
    h3_ref[...] = h3
    logits = jnp.dot(h3, wr_ref[...], preferred_element_type=jnp.float32)
    probs = jax.nn.softmax(logits, axis=-1)
    eio = jax.lax.broadcasted_iota(jnp.int32, logits.shape, 1)
    m1 = jnp.max(logits, axis=-1, keepdims=True)
    idx1 = jnp.min(jnp.where(logits == m1, eio, NE), axis=-1, keepdims=True)
    mask1 = (eio == idx1)
    ml = jnp.where(mask1, -jnp.inf, logits)
    m2 = jnp.max(ml, axis=-1, keepdims=True)
    idx2 = jnp.min(jnp.where(ml == m2, eio, NE), axis=-1, keepdims=True)
    mask2 = (eio == idx2)
    p1 = jnp.sum(jnp.where(mask1, probs, 0.0), axis=-1, keepdims=True)
    p2 = jnp.sum(jnp.where(mask2, probs, 0.0), axis=-1, keepdims=True)
    inv = 1.0 / (p1 + p2)
    wd = jnp.where(mask1, p1 * inv, 0.0) + jnp.where(mask2, p2 * inv, 0.0)
    wd_ref[...] = wd
    ps_ref[...] = jnp.sum(probs, axis=0, keepdims=True)[None]
    cs_ref[...] = jnp.sum(mask1.astype(jnp.float32) + mask2.astype(jnp.float32),
                          axis=0, keepdims=True)[None]


def _k4(h3_ref, w1_ref, b1_ref, w2_ref, b2_ref, wd_ref, x2_ref, out_ref):
    e = pl.program_id(1)
    h = jax.nn.silu(jnp.dot(h3_ref[...], w1_ref[0],
                            preferred_element_type=jnp.float32) + b1_ref[0])
    y = jnp.dot(h, w2_ref[0], preferred_element_type=jnp.float32) + b2_ref[0]
    eio = jax.lax.broadcasted_iota(jnp.int32, wd_ref.shape, 1)
    wcol = jnp.sum(jnp.where(eio == e, wd_ref[...], 0.0), axis=-1,
                   keepdims=True)
    contrib = y * wcol

    @pl.when(e == 0)
    def _():
        out_ref[...] = x2_ref[...] + contrib

    @pl.when(e > 0)
    def _():
        out_ref[...] += contrib


def kernel(x, state, tau1, scale1, tau2, scale2, Win, bin_, rec_w, pos_bias,
           Wout, bout, Wr, W1, b1, W2, b2):
    B, T, D = x.shape
    ds = Wout.shape[0]
    ne = Wr.shape[1]
    fh = W1.shape[2]
    BT = B * T
    nch = T // CK
    BC = B * nch
    xf = x.reshape(BT, D)

    blk1 = min(512, BT)
    n1 = BT // blk1
    f32 = jnp.float32
    u = pl.pallas_call(
        _k1,
        grid=(n1,),
        in_specs=[
            pl.BlockSpec((blk1, D), lambda i: (i, 0)),
            pl.BlockSpec((1, 1), lambda i: (0, 0)),
            pl.BlockSpec((1, D), lambda i: (0, 0)),
            pl.BlockSpec((D, ds), lambda i: (0, 0)),
            pl.BlockSpec((D, ds), lambda i: (0, 0)),
            pl.BlockSpec((1, ds), lambda i: (0, 0)),
            pl.BlockSpec((1, ds), lambda i: (0, 0)),
        ],
        out_specs=pl.BlockSpec((blk1, ds), lambda i: (i, 0)),
        out_shape=jax.ShapeDtypeStruct((BT, ds), f32),
    )(xf, tau1.reshape(1, 1), scale1.reshape(1, D), Win[:, :ds], Win[:, ds:],
      bin_[:ds].reshape(1, ds), bin_[ds:].reshape(1, ds))

    u4 = u.reshape(B, nch, CK, ds)
    ut = jnp.transpose(u4, (3, 2, 0, 1)).reshape(ds, CK, BC)
    uo = u4.reshape(BC, CK, ds)

    fu, hs, new_state = pl.pallas_call(
        functools.partial(_k2, nb=nch),
        grid=(1,),
        in_specs=[
            pl.BlockSpec((ds, CK, BC), lambda i: (0, 0, 0)),
            pl.BlockSpec((BC, CK, ds), lambda i: (0, 0, 0)),
            pl.BlockSpec((ds, 1), lambda i: (0, 0)),
            pl.BlockSpec((ds, 1), lambda i: (0, 0)),
            pl.BlockSpec((B, ds), lambda i: (0, 0)),
        ],
        out_specs=[
            pl.BlockSpec((ds, CK, BC), lambda i: (0, 0, 0)),
            pl.BlockSpec((BC, ds), lambda i: (0, 0)),
            pl.BlockSpec((B, ds), lambda i: (0, 0)),
        ],
        out_shape=[
            jax.ShapeDtypeStruct((ds, CK, BC), f32),
            jax.ShapeDtypeStruct((BC, ds), f32),
            jax.ShapeDtypeStruct((B, ds), f32),
        ],
        scratch_shapes=[pltpu.VMEM((BC, ds), f32)],
    )(ut, uo, rec_w.reshape(ds, 1), pos_bias.reshape(ds, 1), state)

    fu_t = jnp.transpose(fu, (2, 1, 0)).reshape(BT, ds)
    hs_full = jnp.repeat(hs, CK, axis=0)
    a = jax.nn.sigmoid(rec_w + pos_bias)
    jj = (jnp.arange(CK, dtype=f32) + 1.0)[:, None]
    apow = a[None, :] ** jj
    ap_full = jnp.tile(apow, (BC, 1))

    blk3 = min(512, BT)
    n3 = BT // blk3
    x2, h3, wd, psum, csum = pl.pallas_call(
        _k3,
        grid=(n3,),
        in_specs=[
            pl.BlockSpec((blk3, ds), lambda i: (i, 0)),
            pl.BlockSpec((blk3, ds), lambda i: (i, 0)),
            pl.BlockSpec((blk3, ds), lambda i: (i, 0)),
            pl.BlockSpec((blk3, D), lambda i: (i, 0)),
            pl.BlockSpec((1, 1), lambda i: (0, 0)),
            pl.BlockSpec((1, D), lambda i: (0, 0)),
            pl.BlockSpec((ds, D), lambda i: (0, 0)),
            pl.BlockSpec((1, D), lambda i: (0, 0)),
            pl.BlockSpec((D, ne), lambda i: (0, 0)),
        ],
        out_specs=[
            pl.BlockSpec((blk3, D), lambda i: (i, 0)),
            pl.BlockSpec((blk3, D), lambda i: (i, 0)),
            pl.BlockSpec((blk3, ne), lambda i: (i, 0)),
            pl.BlockSpec((1, 1, ne), lambda i: (i, 0, 0)),
            pl.BlockSpec((1, 1, ne), lambda i: (i, 0, 0)),
        ],
        out_shape=[
            jax.ShapeDtypeStruct((BT, D), f32),
            jax.ShapeDtypeStruct((BT, D), f32),
            jax.ShapeDtypeStruct((BT, ne), f32),
            jax.ShapeDtypeStruct((n3, 1, ne), f32),
            jax.ShapeDtypeStruct((n3, 1, ne), f32),
        ],
    )(fu_t, hs_full, ap_full, xf, tau2.reshape(1, 1), scale2.reshape(1, D),
      Wout, bout.reshape(1, D), Wr)

    blk4 = min(512, BT)
    n4 = BT // blk4
    out = pl.pallas_call(
        _k4,
        grid=(n4, ne),
        in_specs=[
            pl.BlockSpec((blk4, D), lambda i, e: (i, 0)),
            pl.BlockSpec((1, D, fh), lambda i, e: (e, 0, 0)),
            pl.BlockSpec((1, 1, fh), lambda i, e: (e, 0, 0)),
            pl.BlockSpec((1, fh, D), lambda i, e: (e, 0, 0)),
            pl.BlockSpec((1, 1, D), lambda i, e: (e, 0, 0)),
            pl.BlockSpec((blk4, ne), lambda i, e: (i, 0)),
            pl.BlockSpec((blk4, D), lambda i, e: (i, 0)),
        ],
        out_specs=pl.BlockSpec((blk4, D), lambda i, e: (i, 0)),
        out_shape=jax.ShapeDtypeStruct((BT, D), f32),
    )(h3, W1, b1.reshape(ne, 1, fh), W2, b2.reshape(ne, 1, D), wd, x2)

    f_i = jnp.sum(csum, axis=(0, 1)) / (BT * NA)
    P_i = jnp.sum(psum, axis=(0, 1)) / BT
    aux = ne * jnp.sum(f_i * P_i)
    return out.reshape(B, T, D), new_state, aux
